# baseline (speedup 1.0000x reference)
import functools

import jax
import jax.numpy as jnp
from jax import lax
from jax.experimental import pallas as pl
from jax.experimental.pallas import tpu as pltpu
from jax.experimental.pallas import tpu_sc as plsc

VOCAB = 10000
D = 100
SEQ = 200
BATCH = 4096
ROWS = BATCH * SEQ

NC = 2
NS = 16
NW = NC * NS
ROWS_PER_W = ROWS // NW
CHUNK = 100
NCHUNK = ROWS_PER_W // CHUNK
SUB_G = 104
DP = 104
DM = 96
DT = 16
TOFF = D - DT


def _body(word_hbm, idx_hbm, posm_hbm, post_hbm, out_hbm,
          table_sh, idx_v, posm_v, post_v, main_v, rows_v,
          semg0, semg1, semo0, semo1, semi0, semi1):
    cid = lax.axis_index("c")
    sid = lax.axis_index("s")
    wid = cid * NS + sid
    rows_per_tile = VOCAB // NS
    pltpu.sync_copy(word_hbm.at[pl.ds(sid * rows_per_tile, rows_per_tile)],
                    table_sh.at[pl.ds(sid * rows_per_tile, rows_per_tile)])
    pltpu.sync_copy(posm_hbm, posm_v)
    pltpu.sync_copy(post_hbm, post_v)
    pltpu.sync_copy(idx_hbm.at[wid, 0], idx_v.at[0])
    plsc.subcore_barrier()
    bbase = wid * (ROWS_PER_W // SEQ)
    semg = (semg0, semg1)
    semo = (semo0, semo1)
    semi = (semi0, semi1)

    def fire_idx(g, p):
        pltpu.async_copy(idx_hbm.at[wid, g], idx_v.at[p], semi[p])

    def wait_idx(p):
        pltpu.make_async_copy(
            idx_hbm.at[wid, 0], idx_v.at[p], semi[p]).wait()

    def fire_gathers(g_p, b):
        pltpu.async_copy(table_sh.at[idx_v.at[g_p]], main_v.at[b], semg[b])

    def wait_gathers(b):
        pltpu.make_async_copy(
            table_sh.at[idx_v.at[0]], main_v.at[b], semg[b]).wait()

    def wait_out(b):
        pltpu.make_async_copy(
            rows_v.at[b], out_hbm.at[0, pl.ds(0, CHUNK)], semo[b]).wait()

    def merge(b):
        p0 = b * CHUNK

        def row_body(r, carry):
            for j in range(6):
                rows_v[b, r, pl.ds(j * 16, 16)] = (
                    main_v[b, r, pl.ds(j * 16, 16)]
                    + posm_v[p0 + r, pl.ds(j * 16, 16)])
            rows_v[b, r, pl.ds(TOFF, 16)] = (
                main_v[b, r, pl.ds(TOFF, 16)] + post_v[p0 + r, :])
            return carry

        lax.fori_loop(0, CHUNK, row_body, 0, unroll=4)

    fire_gathers(0, 0)
    fire_idx(1, 1)

    def pair_body(gg, carry):
        for b in range(2):
            g = gg * 2 + b

            @pl.when(g + 1 < NCHUNK)
            def _():
                wait_idx(1 - b)
                fire_gathers(1 - b, 1 - b)

            wait_gathers(b)

            @pl.when(g + 2 < NCHUNK)
            def _():
                fire_idx(g + 2, b)

            @pl.when(gg >= 1)
            def _():
                wait_out(b)

            merge(b)
            pltpu.async_copy(
                rows_v.at[b],
                out_hbm.at[bbase + gg, pl.ds(b * CHUNK, CHUNK)],
                semo[b])
        return carry

    lax.fori_loop(0, NCHUNK // 2, pair_body, 0)
    wait_out(0)
    wait_out(1)


@functools.partial(
    pl.kernel,
    out_type=jax.ShapeDtypeStruct((BATCH, SEQ, D), jnp.float32),
    mesh=plsc.VectorSubcoreMesh(core_axis_name="c", subcore_axis_name="s"),
    scratch_types=[
        pltpu.VMEM_SHARED((VOCAB, DP), jnp.float32),
        pltpu.VMEM((2, SUB_G), jnp.int32),
        pltpu.VMEM((SEQ, DM), jnp.float32),
        pltpu.VMEM((SEQ, DT), jnp.float32),
        pltpu.VMEM((2, SUB_G, DP), jnp.float32),
        pltpu.VMEM((2, CHUNK, D), jnp.float32),
        pltpu.SemaphoreType.DMA,
        pltpu.SemaphoreType.DMA,
        pltpu.SemaphoreType.DMA,
        pltpu.SemaphoreType.DMA,
        pltpu.SemaphoreType.DMA,
        pltpu.SemaphoreType.DMA,
    ],
    compiler_params=pltpu.CompilerParams(use_tc_tiling_on_sc=False),
)
def _embed_kernel(word_hbm, idx_hbm, posm_hbm, post_hbm, out_hbm,
                  table_sh, idx_v, posm_v, post_v, main_v, rows_v,
                  semg0, semg1, semo0, semo1, semi0, semi1):
    _body(word_hbm, idx_hbm, posm_hbm, post_hbm, out_hbm,
          table_sh, idx_v, posm_v, post_v, main_v, rows_v,
          semg0, semg1, semo0, semo1, semi0, semi1)


def kernel(inputs, word_table, pos_table):
    idx = inputs.reshape(ROWS // CHUNK, CHUNK).astype(jnp.int32)
    idx = jnp.concatenate([idx, jnp.roll(idx, -1, axis=0)[:, :4]], axis=1)
    idx = idx.reshape(NW, NCHUNK, SUB_G)
    word_pad = jnp.pad(word_table, ((0, 0), (0, DP - D)))
    return _embed_kernel(word_pad, idx, pos_table[:, :DM],
                         pos_table[:, TOFF:])


# speedup vs baseline: 1.6694x; 1.6694x over previous
import functools

import jax
import jax.numpy as jnp
from jax import lax
from jax.experimental import pallas as pl
from jax.experimental.pallas import tpu as pltpu
from jax.experimental.pallas import tpu_sc as plsc

VOCAB = 10000
D = 100
SEQ = 200
BATCH = 4096
ROWS = BATCH * SEQ

NC = 2
NS = 16
NW = NC * NS
ROWS_PER_W = ROWS // NW
CHUNK = 100
NCHUNK = ROWS_PER_W // CHUNK
SUB_G = 104
DP = 104
DM = 96
DT = 16
TOFF = D - DT


def _body(word_hbm, idx_hbm, posm_hbm, post_hbm, out_hbm,
          table_sh, idx_v, posm_v, post_v, main_v, rows_v,
          semg0, semg1, semo0, semo1, semi0, semi1):
    cid = lax.axis_index("c")
    sid = lax.axis_index("s")
    wid = cid * NS + sid
    rows_per_tile = VOCAB // NS
    pltpu.sync_copy(word_hbm.at[pl.ds(sid * rows_per_tile, rows_per_tile)],
                    table_sh.at[pl.ds(sid * rows_per_tile, rows_per_tile)])
    pltpu.sync_copy(posm_hbm, posm_v)
    pltpu.sync_copy(post_hbm, post_v)
    pltpu.sync_copy(idx_hbm.at[wid, 0], idx_v.at[0])
    plsc.subcore_barrier()
    bbase = wid * (ROWS_PER_W // SEQ)
    semg = (semg0, semg1)
    semo = (semo0, semo1)
    semi = (semi0, semi1)

    def fire_idx(g, p):
        pltpu.async_copy(idx_hbm.at[wid, g], idx_v.at[p], semi[p])

    def wait_idx(p):
        pltpu.make_async_copy(
            idx_hbm.at[wid, 0], idx_v.at[p], semi[p]).wait()

    def fire_gathers(g_p, b):
        pltpu.async_copy(table_sh.at[idx_v.at[g_p]], main_v.at[b], semg[b])

    def wait_gathers(b):
        pltpu.make_async_copy(
            table_sh.at[idx_v.at[0]], main_v.at[b], semg[b]).wait()

    def wait_out(b):
        pltpu.make_async_copy(
            rows_v.at[b], out_hbm.at[0, pl.ds(0, CHUNK)], semo[b]).wait()

    def merge(b):
        p0 = b * CHUNK

        @plsc.parallel_loop(0, CHUNK, unroll=4)
        def row_body(r):
            for j in range(6):
                rows_v[b, r, pl.ds(j * 16, 16)] = (
                    main_v[b, r, pl.ds(j * 16, 16)]
                    + posm_v[p0 + r, pl.ds(j * 16, 16)])
            rows_v[b, r, pl.ds(TOFF, 16)] = (
                main_v[b, r, pl.ds(TOFF, 16)] + post_v[p0 + r, :])

    fire_gathers(0, 0)
    fire_idx(1, 1)

    def pair_body(gg, carry):
        for b in range(2):
            g = gg * 2 + b

            @pl.when(g + 1 < NCHUNK)
            def _():
                wait_idx(1 - b)
                fire_gathers(1 - b, 1 - b)

            wait_gathers(b)

            @pl.when(g + 2 < NCHUNK)
            def _():
                fire_idx(g + 2, b)

            @pl.when(gg >= 1)
            def _():
                wait_out(b)

            merge(b)
            pltpu.async_copy(
                rows_v.at[b],
                out_hbm.at[bbase + gg, pl.ds(b * CHUNK, CHUNK)],
                semo[b])
        return carry

    lax.fori_loop(0, NCHUNK // 2, pair_body, 0)
    wait_out(0)
    wait_out(1)


@functools.partial(
    pl.kernel,
    out_type=jax.ShapeDtypeStruct((BATCH, SEQ, D), jnp.float32),
    mesh=plsc.VectorSubcoreMesh(core_axis_name="c", subcore_axis_name="s"),
    scratch_types=[
        pltpu.VMEM_SHARED((VOCAB, DP), jnp.float32),
        pltpu.VMEM((2, SUB_G), jnp.int32),
        pltpu.VMEM((SEQ, DM), jnp.float32),
        pltpu.VMEM((SEQ, DT), jnp.float32),
        pltpu.VMEM((2, SUB_G, DP), jnp.float32),
        pltpu.VMEM((2, CHUNK, D), jnp.float32),
        pltpu.SemaphoreType.DMA,
        pltpu.SemaphoreType.DMA,
        pltpu.SemaphoreType.DMA,
        pltpu.SemaphoreType.DMA,
        pltpu.SemaphoreType.DMA,
        pltpu.SemaphoreType.DMA,
    ],
    compiler_params=pltpu.CompilerParams(use_tc_tiling_on_sc=False),
)
def _embed_kernel(word_hbm, idx_hbm, posm_hbm, post_hbm, out_hbm,
                  table_sh, idx_v, posm_v, post_v, main_v, rows_v,
                  semg0, semg1, semo0, semo1, semi0, semi1):
    _body(word_hbm, idx_hbm, posm_hbm, post_hbm, out_hbm,
          table_sh, idx_v, posm_v, post_v, main_v, rows_v,
          semg0, semg1, semo0, semo1, semi0, semi1)


def kernel(inputs, word_table, pos_table):
    idx = inputs.reshape(ROWS // CHUNK, CHUNK).astype(jnp.int32)
    idx = jnp.concatenate([idx, jnp.roll(idx, -1, axis=0)[:, :4]], axis=1)
    idx = idx.reshape(NW, NCHUNK, SUB_G)
    word_pad = jnp.pad(word_table, ((0, 0), (0, DP - D)))
    return _embed_kernel(word_pad, idx, pos_table[:, :DM],
                         pos_table[:, TOFF:])
